# baseline (speedup 1.0000x reference)
import functools
import math

import jax
import jax.numpy as jnp
from jax import lax
from jax.experimental import pallas as pl
from jax.experimental.pallas import tpu as pltpu
from jax.experimental.pallas import tpu_sc as plsc

D_M = 64
SCALE = math.sqrt(D_M)
LANES = 16
CHUNK = 100
NBUF = 4


@functools.lru_cache(maxsize=None)
def _build(rows: int, cols: int, num_cores: int, num_subcores: int):
    nw = num_cores * num_subcores
    n_ch = rows * cols // (nw * CHUNK)
    ch_per_row = cols // CHUNK
    mesh = plsc.VectorSubcoreMesh(core_axis_name="c", subcore_axis_name="s")
    assert n_ch % NBUF == 0 and n_ch >= 2 * NBUF

    @functools.partial(
        pl.kernel,
        mesh=mesh,
        out_type=jax.ShapeDtypeStruct((rows * cols, 2 * D_M), jnp.float32),
        compiler_params=pltpu.CompilerParams(use_tc_tiling_on_sc=False),
        scratch_types=[
            pltpu.VMEM((n_ch, CHUNK), jnp.int32),
            pltpu.VMEM((NBUF, CHUNK, D_M), jnp.float32),
            pltpu.VMEM((NBUF, CHUNK, 2 * D_M), jnp.float32),
            [pltpu.SemaphoreType.DMA] * NBUF,
            [pltpu.SemaphoreType.DMA] * NBUF,
        ],
    )
    def emb(x_hbm, tab_hbm, out_hbm, idx_v, gbuf, sbuf, gsems, ssems):
        wid = lax.axis_index("s") * num_cores + lax.axis_index("c")
        base = wid * n_ch
        pltpu.sync_copy(x_hbm.at[pl.ds(base, n_ch)], idx_v)

        for b in range(NBUF):
            pltpu.async_copy(tab_hbm.at[idx_v.at[b]], gbuf.at[b], gsems[b])

        def scale(b):
            def row(r, c2):
                for c in range(D_M // LANES):
                    sl = pl.ds(c * LANES, LANES)
                    sbuf[b, r, sl] = gbuf[b, r, sl] * SCALE
                return c2

            lax.fori_loop(0, CHUNK, row, 0)

        @pl.loop(0, n_ch, step=NBUF)
        def outer(j0):
            for b in range(NBUF):
                k = j0 + b
                pltpu.make_async_copy(
                    tab_hbm.at[pl.ds(0, CHUNK)], gbuf.at[b], gsems[b]
                ).wait()
                @pl.when(j0 > 0)
                def _():
                    pltpu.make_async_copy(
                        sbuf.at[b], out_hbm.at[pl.ds(0, CHUNK)], ssems[b]
                    ).wait()

                scale(b)
                @pl.when(k + NBUF < n_ch)
                def _():
                    pltpu.async_copy(
                        tab_hbm.at[idx_v.at[k + NBUF]], gbuf.at[b], gsems[b]
                    )

                pltpu.async_copy(
                    sbuf.at[b],
                    out_hbm.at[pl.ds((base + k) * CHUNK, CHUNK)],
                    ssems[b],
                )

        for b in range(NBUF):
            pltpu.make_async_copy(
                sbuf.at[b], out_hbm.at[pl.ds(0, CHUNK)], ssems[b]
            ).wait()

    return emb


def kernel(x, table):
    rows, cols = x.shape
    info = plsc.get_sparse_core_info()
    nw = info.num_cores * info.num_subcores
    assert (rows * cols) % (nw * CHUNK) == 0 and cols % CHUNK == 0
    n_ch = rows * cols // (nw * CHUNK)
    xf = x.reshape(nw * n_ch, CHUNK).astype(jnp.int32)
    emb = _build(rows, cols, info.num_cores, info.num_subcores)
    out2 = emb(xf, table)
    return out2[:, :D_M].reshape(rows, cols, D_M)


# speedup vs baseline: 1.3679x; 1.3679x over previous
import functools
import math

import jax
import jax.numpy as jnp
from jax import lax
from jax.experimental import pallas as pl
from jax.experimental.pallas import tpu as pltpu
from jax.experimental.pallas import tpu_sc as plsc

D_M = 64
SCALE = math.sqrt(D_M)
LANES = 16
CHUNK = 100
NBUF = 4


@functools.lru_cache(maxsize=None)
def _build(rows: int, cols: int, num_cores: int, num_subcores: int):
    nw = num_cores * num_subcores
    n_ch = rows * cols // (nw * CHUNK)
    ch_per_row = cols // CHUNK
    mesh = plsc.VectorSubcoreMesh(core_axis_name="c", subcore_axis_name="s")
    assert n_ch % NBUF == 0 and n_ch >= 2 * NBUF

    @functools.partial(
        pl.kernel,
        mesh=mesh,
        out_type=jax.ShapeDtypeStruct((rows * cols, 2 * D_M), jnp.float32),
        compiler_params=pltpu.CompilerParams(use_tc_tiling_on_sc=False),
        scratch_types=[
            pltpu.VMEM((n_ch, CHUNK), jnp.int32),
            pltpu.VMEM((NBUF, CHUNK, D_M), jnp.float32),
            pltpu.VMEM((NBUF, CHUNK, D_M), jnp.float32),
            [pltpu.SemaphoreType.DMA] * NBUF,
            [pltpu.SemaphoreType.DMA] * NBUF,
        ],
    )
    def emb(x_hbm, tab_hbm, out_hbm, idx_v, gbuf, sbuf, gsems, ssems):
        wid = lax.axis_index("s") * num_cores + lax.axis_index("c")
        base = wid * n_ch
        pltpu.sync_copy(x_hbm.at[pl.ds(base, n_ch)], idx_v)

        for b in range(NBUF):
            pltpu.async_copy(tab_hbm.at[idx_v.at[b]], gbuf.at[b], gsems[b])

        def scale(b):
            def row(r, c2):
                for c in range(D_M // LANES):
                    sl = pl.ds(c * LANES, LANES)
                    sbuf[b, r, sl] = gbuf[b, r, sl] * SCALE
                return c2

            lax.fori_loop(0, CHUNK, row, 0)

        @pl.loop(0, n_ch, step=NBUF)
        def outer(j0):
            for b in range(NBUF):
                k = j0 + b
                pltpu.make_async_copy(
                    tab_hbm.at[pl.ds(0, CHUNK)], gbuf.at[b], gsems[b]
                ).wait()
                @pl.when(j0 > 0)
                def _():
                    pltpu.make_async_copy(
                        sbuf.at[b],
                        out_hbm.at[pl.ds(0, CHUNK), pl.ds(0, D_M)],
                        ssems[b],
                    ).wait()

                scale(b)
                @pl.when(k + NBUF < n_ch)
                def _():
                    pltpu.async_copy(
                        tab_hbm.at[idx_v.at[k + NBUF]], gbuf.at[b], gsems[b]
                    )

                pltpu.async_copy(
                    sbuf.at[b],
                    out_hbm.at[pl.ds((base + k) * CHUNK, CHUNK), pl.ds(0, D_M)],
                    ssems[b],
                )

        for b in range(NBUF):
            pltpu.make_async_copy(
                sbuf.at[b],
                out_hbm.at[pl.ds(0, CHUNK), pl.ds(0, D_M)],
                ssems[b],
            ).wait()

    return emb


def kernel(x, table):
    rows, cols = x.shape
    info = plsc.get_sparse_core_info()
    nw = info.num_cores * info.num_subcores
    assert (rows * cols) % (nw * CHUNK) == 0 and cols % CHUNK == 0
    n_ch = rows * cols // (nw * CHUNK)
    xf = x.reshape(nw * n_ch, CHUNK).astype(jnp.int32)
    emb = _build(rows, cols, info.num_cores, info.num_subcores)
    out2 = emb(xf, table)
    return out2[:, :D_M].reshape(rows, cols, D_M)
